# Initial kernel scaffold; baseline (speedup 1.0000x reference)
import jax
import jax.numpy as jnp
from jax.experimental import pallas as pl


def kernel(router_weights, router_logits, num_experts_per_tok, non_pad_token):
    raise NotImplementedError("write your pallas kernel here")


# speedup vs baseline: 1.5765x; 1.5765x over previous
import functools

import jax
import jax.numpy as jnp
from jax import lax
from jax.experimental import pallas as pl
from jax.experimental.pallas import tpu as pltpu
from jax.experimental.pallas import tpu_sc as plsc

L_LAYERS = 16
SEQ = 4096
E = 64
NC = 2
NS = 16
LANES = 16

TOK_PER_TEC = SEQ // NC
CHUNK = 512
GROUPS = CHUNK // LANES
N_CHUNKS = TOK_PER_TEC // CHUNK

_mesh = plsc.VectorSubcoreMesh(
    core_axis_name="c", subcore_axis_name="s", num_cores=NC, num_subcores=NS
)


@functools.partial(
    pl.kernel,
    out_type=(
        jax.ShapeDtypeStruct((NS, NC, E, LANES), jnp.float32),
        jax.ShapeDtypeStruct((NS, NC, E, LANES), jnp.float32),
    ),
    mesh=_mesh,
    scratch_types=[
        pltpu.VMEM((CHUNK * E,), jnp.float32),
        pltpu.VMEM((E, LANES), jnp.float32),
        pltpu.VMEM((E, LANES), jnp.float32),
    ],
    compiler_params=pltpu.CompilerParams(
        use_tc_tiling_on_sc=False, needs_layout_passes=False
    ),
)
def _sc_count_gsum(w_hbm, cnt_out, gsum_out, chunk_vm, cnt_vm, gsum_vm):
    c = lax.axis_index("c")
    s = lax.axis_index("s")
    layer = s
    elem0 = (c * TOK_PER_TEC) * E

    zero = jnp.zeros((LANES,), jnp.float32)
    for e in range(E):
        cnt_vm[e] = zero
        gsum_vm[e] = zero

    iota64 = lax.iota(jnp.int32, LANES) * E
    neg = jnp.full((LANES,), -jnp.inf, jnp.float32)

    def chunk_body(ci, _):
        pltpu.sync_copy(
            w_hbm.at[layer, pl.ds(elem0 + ci * (CHUNK * E), CHUNK * E)],
            chunk_vm,
        )

        def group_body(g, _):
            base = g * (LANES * E) + iota64
            m1 = neg
            m2 = neg
            for e in range(E):
                v = plsc.load_gather(chunk_vm, [base + e])
                m2 = jnp.maximum(m2, jnp.minimum(m1, v))
                m1 = jnp.maximum(m1, v)
            for e in range(E):
                v = plsc.load_gather(chunk_vm, [base + e])
                cnt_vm[e] = cnt_vm[e] + jnp.where(v >= m2, 1.0, 0.0)
                gsum_vm[e] = gsum_vm[e] + v
            return 0

        lax.fori_loop(0, GROUPS, group_body, 0)
        return 0

    lax.fori_loop(0, N_CHUNKS, chunk_body, 0)

    pltpu.sync_copy(cnt_vm, cnt_out.at[s, c])
    pltpu.sync_copy(gsum_vm, gsum_out.at[s, c])


def _combine_body(scale_ref, cnt_ref, gsum_ref, out_ref):
    x = cnt_ref[...]
    y = gsum_ref[...]
    x1 = x[:, : E * LANES] + x[:, E * LANES :]
    y1 = y[:, : E * LANES] + y[:, E * LANES :]
    i = lax.broadcasted_iota(jnp.int32, (E * LANES, E), 0)
    j = lax.broadcasted_iota(jnp.int32, (E * LANES, E), 1)
    sel = jnp.where(i // LANES == j, 1.0, 0.0)
    cs = jnp.dot(x1, sel, preferred_element_type=jnp.float32)
    gs = jnp.dot(y1, sel, preferred_element_type=jnp.float32)
    out_ref[0, 0] = jnp.sum(cs * gs) * scale_ref[0]


def kernel(router_weights, router_logits, num_experts_per_tok, non_pad_token):
    del router_logits
    w_flat = router_weights.reshape(L_LAYERS, SEQ * E)
    cnt_p, gsum_p = _sc_count_gsum(w_flat)
    valid = jnp.maximum(non_pad_token, 1)
    scale = (E / (valid * num_experts_per_tok)) / valid
    scale = jnp.asarray(scale, jnp.float32).reshape((1,))
    cnt2 = cnt_p.reshape(L_LAYERS, NC * E * LANES)
    gsum2 = gsum_p.reshape(L_LAYERS, NC * E * LANES)
    out = pl.pallas_call(
        _combine_body,
        out_shape=jax.ShapeDtypeStruct((1, 1), jnp.float32),
        in_specs=[
            pl.BlockSpec(memory_space=pltpu.SMEM),
            pl.BlockSpec(memory_space=pltpu.VMEM),
            pl.BlockSpec(memory_space=pltpu.VMEM),
        ],
        out_specs=pl.BlockSpec(memory_space=pltpu.SMEM),
    )(scale, cnt2, gsum2)
    return out[0, 0]


# speedup vs baseline: 2.1658x; 1.3738x over previous
import functools

import jax
import jax.numpy as jnp
from jax import lax
from jax.experimental import pallas as pl
from jax.experimental.pallas import tpu as pltpu
from jax.experimental.pallas import tpu_sc as plsc

L_LAYERS = 16
SEQ = 4096
E = 64
NC = 2
NS = 16
LANES = 16

TOK_PER_TEC = SEQ // NC
CHUNK = 1024
GROUPS = CHUNK // LANES
N_CHUNKS = TOK_PER_TEC // CHUNK
N_CHAINS = 8
BANK = 8

_mesh = plsc.VectorSubcoreMesh(
    core_axis_name="c", subcore_axis_name="s", num_cores=NC, num_subcores=NS
)


@functools.partial(
    pl.kernel,
    out_type=(
        jax.ShapeDtypeStruct((NS, NC, E, LANES), jnp.float32),
        jax.ShapeDtypeStruct((NS, NC, E, LANES), jnp.float32),
    ),
    mesh=_mesh,
    scratch_types=[
        pltpu.VMEM((CHUNK * E,), jnp.float32),
        pltpu.VMEM((GROUPS, LANES), jnp.float32),
        pltpu.VMEM((E, LANES), jnp.float32),
        pltpu.VMEM((E, LANES), jnp.float32),
    ],
    compiler_params=pltpu.CompilerParams(
        use_tc_tiling_on_sc=False, needs_layout_passes=False
    ),
)
def _sc_count_gsum(w_hbm, cnt_out, gsum_out, chunk_vm, m2_vm, cnt_vm, gsum_vm):
    c = lax.axis_index("c")
    s = lax.axis_index("s")
    elem0 = (c * TOK_PER_TEC) * E

    iota64 = lax.iota(jnp.int32, LANES) * E
    neg = jnp.full((LANES,), -jnp.inf, jnp.float32)
    zero = jnp.zeros((LANES,), jnp.float32)

    def merge(a, b):
        m1a, m2a = a
        m1b, m2b = b
        return (
            jnp.maximum(m1a, m1b),
            jnp.maximum(jnp.minimum(m1a, m1b), jnp.maximum(m2a, m2b)),
        )

    for ci in range(N_CHUNKS):
        pltpu.sync_copy(
            w_hbm.at[s, pl.ds(elem0 + ci * (CHUNK * E), CHUNK * E)],
            chunk_vm,
        )

        @plsc.parallel_loop(0, GROUPS, 1, unroll=2)
        def _pass_a(g):
            base = g * (LANES * E) + iota64
            m1s = [neg] * N_CHAINS
            m2s = [neg] * N_CHAINS
            for e in range(E):
                j = e % N_CHAINS
                v = plsc.load_gather(chunk_vm, [base + e])
                m2s[j] = jnp.maximum(m2s[j], jnp.minimum(m1s[j], v))
                m1s[j] = jnp.maximum(m1s[j], v)
            ps = list(zip(m1s, m2s))
            while len(ps) > 1:
                ps = [merge(ps[i], ps[i + 1]) for i in range(0, len(ps), 2)]
            m2_vm[g] = ps[0][1]

        for b in range(E // BANK):
            es = list(range(b * BANK, (b + 1) * BANK))
            if ci == 0:
                carry = (tuple([zero] * BANK), tuple([zero] * BANK))
            else:
                carry = (
                    tuple(cnt_vm[e] for e in es),
                    tuple(gsum_vm[e] for e in es),
                )

            @plsc.parallel_loop(0, GROUPS, 1, unroll=2, carry=carry)
            def _pass_b(g, regs, es=es):
                cnts, gsums = regs
                cnts, gsums = list(cnts), list(gsums)
                base = g * (LANES * E) + iota64
                m2 = m2_vm[g]
                for j, e in enumerate(es):
                    v = plsc.load_gather(chunk_vm, [base + e])
                    cnts[j] = cnts[j] + jnp.where(v >= m2, 1.0, 0.0)
                    gsums[j] = gsums[j] + v
                return (tuple(cnts), tuple(gsums))

            cnts_f, gsums_f = _pass_b
            for j, e in enumerate(es):
                cnt_vm[e] = cnts_f[j]
                gsum_vm[e] = gsums_f[j]

    pltpu.sync_copy(cnt_vm, cnt_out.at[s, c])
    pltpu.sync_copy(gsum_vm, gsum_out.at[s, c])


def _combine_body(scale_ref, cnt_ref, gsum_ref, out_ref):
    x = cnt_ref[...]
    y = gsum_ref[...]
    x1 = x[:, : E * LANES] + x[:, E * LANES :]
    y1 = y[:, : E * LANES] + y[:, E * LANES :]
    i = lax.broadcasted_iota(jnp.int32, (E * LANES, E), 0)
    j = lax.broadcasted_iota(jnp.int32, (E * LANES, E), 1)
    sel = jnp.where(i // LANES == j, 1.0, 0.0)
    cs = jnp.dot(x1, sel, preferred_element_type=jnp.float32)
    gs = jnp.dot(y1, sel, preferred_element_type=jnp.float32)
    out_ref[0, 0] = jnp.sum(cs * gs) * scale_ref[0]


def kernel(router_weights, router_logits, num_experts_per_tok, non_pad_token):
    del router_logits
    w_flat = router_weights.reshape(L_LAYERS, SEQ * E)
    cnt_p, gsum_p = _sc_count_gsum(w_flat)
    valid = jnp.maximum(non_pad_token, 1)
    scale = (E / (valid * num_experts_per_tok)) / valid
    scale = jnp.asarray(scale, jnp.float32).reshape((1,))
    cnt2 = cnt_p.reshape(L_LAYERS, NC * E * LANES)
    gsum2 = gsum_p.reshape(L_LAYERS, NC * E * LANES)
    out = pl.pallas_call(
        _combine_body,
        out_shape=jax.ShapeDtypeStruct((1, 1), jnp.float32),
        in_specs=[
            pl.BlockSpec(memory_space=pltpu.SMEM),
            pl.BlockSpec(memory_space=pltpu.VMEM),
            pl.BlockSpec(memory_space=pltpu.VMEM),
        ],
        out_specs=pl.BlockSpec(memory_space=pltpu.SMEM),
    )(scale, cnt2, gsum2)
    return out[0, 0]


# speedup vs baseline: 3.8064x; 1.7575x over previous
import functools

import jax
import jax.numpy as jnp
from jax import lax
from jax.experimental import pallas as pl
from jax.experimental.pallas import tpu as pltpu
from jax.experimental.pallas import tpu_sc as plsc

L_LAYERS = 16
SEQ = 4096
E = 64
NC = 2
NS = 16
LANES = 16

TOK_PER_TEC = SEQ // NC
CHUNK = 1024
GROUPS = CHUNK // LANES
N_CHUNKS = TOK_PER_TEC // CHUNK
N_CHAINS = 8
BANK = 8

_mesh = plsc.VectorSubcoreMesh(
    core_axis_name="c", subcore_axis_name="s", num_cores=NC, num_subcores=NS
)


@functools.partial(
    pl.kernel,
    out_type=(
        jax.ShapeDtypeStruct((NS, NC, E, LANES), jnp.float32),
        jax.ShapeDtypeStruct((NS, NC, E, LANES), jnp.float32),
    ),
    mesh=_mesh,
    scratch_types=[
        pltpu.VMEM((CHUNK, E + 1), jnp.float32),
        pltpu.VMEM((GROUPS, LANES), jnp.float32),
        pltpu.VMEM((E, LANES), jnp.float32),
        pltpu.VMEM((E, LANES), jnp.float32),
    ],
    compiler_params=pltpu.CompilerParams(
        use_tc_tiling_on_sc=False, needs_layout_passes=False
    ),
)
def _sc_count_gsum(w_hbm, cnt_out, gsum_out, chunk_vm, m2_vm, cnt_vm, gsum_vm):
    c = lax.axis_index("c")
    s = lax.axis_index("s")
    tok0 = c * TOK_PER_TEC

    iota = lax.iota(jnp.int32, LANES)
    neg = jnp.full((LANES,), -jnp.inf, jnp.float32)
    zero = jnp.zeros((LANES,), jnp.float32)
    e_splats = [jnp.full((LANES,), e, jnp.int32) for e in range(E)]

    def merge(a, b):
        m1a, m2a = a
        m1b, m2b = b
        return (
            jnp.maximum(m1a, m1b),
            jnp.maximum(jnp.minimum(m1a, m1b), jnp.maximum(m2a, m2b)),
        )

    for ci in range(N_CHUNKS):
        pltpu.sync_copy(
            w_hbm.at[s, pl.ds(tok0 + ci * CHUNK, CHUNK), :],
            chunk_vm.at[:, pl.ds(0, E)],
        )

        @plsc.parallel_loop(0, GROUPS, 1, unroll=2)
        def _pass_a(g):
            tok = g * LANES + iota
            m1s = [neg] * N_CHAINS
            m2s = [neg] * N_CHAINS
            for e in range(E):
                j = e % N_CHAINS
                v = plsc.load_gather(chunk_vm, [tok, e_splats[e]])
                m2s[j] = jnp.maximum(m2s[j], jnp.minimum(m1s[j], v))
                m1s[j] = jnp.maximum(m1s[j], v)
            ps = list(zip(m1s, m2s))
            while len(ps) > 1:
                ps = [merge(ps[i], ps[i + 1]) for i in range(0, len(ps), 2)]
            m2_vm[g] = ps[0][1]

        for b in range(E // BANK):
            es = list(range(b * BANK, (b + 1) * BANK))
            if ci == 0:
                carry = (tuple([zero] * BANK), tuple([zero] * BANK))
            else:
                carry = (
                    tuple(cnt_vm[e] for e in es),
                    tuple(gsum_vm[e] for e in es),
                )

            @plsc.parallel_loop(0, GROUPS, 1, unroll=2, carry=carry)
            def _pass_b(g, regs, es=es):
                cnts, gsums = regs
                cnts, gsums = list(cnts), list(gsums)
                tok = g * LANES + iota
                m2 = m2_vm[g]
                for j, e in enumerate(es):
                    v = plsc.load_gather(chunk_vm, [tok, e_splats[e]])
                    cnts[j] = cnts[j] + jnp.where(v >= m2, 1.0, 0.0)
                    gsums[j] = gsums[j] + v
                return (tuple(cnts), tuple(gsums))

            cnts_f, gsums_f = _pass_b
            for j, e in enumerate(es):
                cnt_vm[e] = cnts_f[j]
                gsum_vm[e] = gsums_f[j]

    pltpu.sync_copy(cnt_vm, cnt_out.at[s, c])
    pltpu.sync_copy(gsum_vm, gsum_out.at[s, c])


def _combine_body(scale_ref, cnt_ref, gsum_ref, out_ref):
    x = cnt_ref[...]
    y = gsum_ref[...]
    x1 = x[:, : E * LANES] + x[:, E * LANES :]
    y1 = y[:, : E * LANES] + y[:, E * LANES :]
    i = lax.broadcasted_iota(jnp.int32, (E * LANES, E), 0)
    j = lax.broadcasted_iota(jnp.int32, (E * LANES, E), 1)
    sel = jnp.where(i // LANES == j, 1.0, 0.0)
    cs = jnp.dot(x1, sel, preferred_element_type=jnp.float32)
    gs = jnp.dot(y1, sel, preferred_element_type=jnp.float32)
    out_ref[0, 0] = jnp.sum(cs * gs) * scale_ref[0]


def kernel(router_weights, router_logits, num_experts_per_tok, non_pad_token):
    del router_logits
    cnt_p, gsum_p = _sc_count_gsum(router_weights)
    valid = jnp.maximum(non_pad_token, 1)
    scale = (E / (valid * num_experts_per_tok)) / valid
    scale = jnp.asarray(scale, jnp.float32).reshape((1,))
    cnt2 = cnt_p.reshape(L_LAYERS, NC * E * LANES)
    gsum2 = gsum_p.reshape(L_LAYERS, NC * E * LANES)
    out = pl.pallas_call(
        _combine_body,
        out_shape=jax.ShapeDtypeStruct((1, 1), jnp.float32),
        in_specs=[
            pl.BlockSpec(memory_space=pltpu.SMEM),
            pl.BlockSpec(memory_space=pltpu.VMEM),
            pl.BlockSpec(memory_space=pltpu.VMEM),
        ],
        out_specs=pl.BlockSpec(memory_space=pltpu.SMEM),
    )(scale, cnt2, gsum2)
    return out[0, 0]


# speedup vs baseline: 3.8284x; 1.0058x over previous
import functools

import jax
import jax.numpy as jnp
from jax import lax
from jax.experimental import pallas as pl
from jax.experimental.pallas import tpu as pltpu
from jax.experimental.pallas import tpu_sc as plsc

L_LAYERS = 16
SEQ = 4096
E = 64
NC = 2
NS = 16
LANES = 16

TOK_PER_TEC = SEQ // NC
CHUNK = 1024
GROUPS = CHUNK // LANES
N_CHUNKS = TOK_PER_TEC // CHUNK
N_CHAINS = 8
BANK = 8

_mesh = plsc.VectorSubcoreMesh(
    core_axis_name="c", subcore_axis_name="s", num_cores=NC, num_subcores=NS
)


@functools.partial(
    pl.kernel,
    out_type=(
        jax.ShapeDtypeStruct((NS, NC, E, LANES), jnp.float32),
        jax.ShapeDtypeStruct((NS, NC, E, LANES), jnp.float32),
    ),
    mesh=_mesh,
    scratch_types=[
        pltpu.VMEM((CHUNK, E + 1), jnp.float32),
        pltpu.VMEM((GROUPS, LANES), jnp.float32),
        pltpu.VMEM((E, LANES), jnp.float32),
        pltpu.VMEM((E, LANES), jnp.float32),
    ],
    compiler_params=pltpu.CompilerParams(
        use_tc_tiling_on_sc=False, needs_layout_passes=False
    ),
)
def _sc_count_gsum(w_hbm, cnt_out, gsum_out, chunk_vm, m2_vm, cnt_vm, gsum_vm):
    c = lax.axis_index("c")
    s = lax.axis_index("s")
    tok0 = c * TOK_PER_TEC

    iota = lax.iota(jnp.int32, LANES)
    neg = jnp.full((LANES,), -jnp.inf, jnp.float32)
    zero = jnp.zeros((LANES,), jnp.float32)
    e_splats = [jnp.full((LANES,), e, jnp.int32) for e in range(E)]

    def merge(a, b):
        m1a, m2a = a
        m1b, m2b = b
        return (
            jnp.maximum(m1a, m1b),
            jnp.maximum(jnp.minimum(m1a, m1b), jnp.maximum(m2a, m2b)),
        )

    for ci in range(N_CHUNKS):
        pltpu.sync_copy(
            w_hbm.at[s, pl.ds(tok0 + ci * CHUNK, CHUNK), :],
            chunk_vm.at[:, pl.ds(0, E)],
        )

        @plsc.parallel_loop(0, GROUPS, 1, unroll=4)
        def _pass_a(g):
            tok = g * LANES + iota
            m1s = [neg] * N_CHAINS
            m2s = [neg] * N_CHAINS
            for e in range(E):
                j = e % N_CHAINS
                v = plsc.load_gather(chunk_vm, [tok, e_splats[e]])
                m2s[j] = jnp.maximum(m2s[j], jnp.minimum(m1s[j], v))
                m1s[j] = jnp.maximum(m1s[j], v)
            ps = list(zip(m1s, m2s))
            while len(ps) > 1:
                ps = [merge(ps[i], ps[i + 1]) for i in range(0, len(ps), 2)]
            m2_vm[g] = ps[0][1]

        for b in range(E // BANK):
            es = list(range(b * BANK, (b + 1) * BANK))
            if ci == 0:
                carry = (tuple([zero] * BANK), tuple([zero] * BANK))
            else:
                carry = (
                    tuple(cnt_vm[e] for e in es),
                    tuple(gsum_vm[e] for e in es),
                )

            @plsc.parallel_loop(0, GROUPS, 1, unroll=4, carry=carry)
            def _pass_b(g, regs, es=es):
                cnts, gsums = regs
                cnts, gsums = list(cnts), list(gsums)
                tok = g * LANES + iota
                m2 = m2_vm[g]
                for j, e in enumerate(es):
                    v = plsc.load_gather(chunk_vm, [tok, e_splats[e]])
                    cnts[j] = cnts[j] + jnp.where(v >= m2, 1.0, 0.0)
                    gsums[j] = gsums[j] + v
                return (tuple(cnts), tuple(gsums))

            cnts_f, gsums_f = _pass_b
            for j, e in enumerate(es):
                cnt_vm[e] = cnts_f[j]
                gsum_vm[e] = gsums_f[j]

    pltpu.sync_copy(cnt_vm, cnt_out.at[s, c])
    pltpu.sync_copy(gsum_vm, gsum_out.at[s, c])


def _combine_body(scale_ref, cnt_ref, gsum_ref, out_ref):
    x = cnt_ref[...]
    y = gsum_ref[...]
    x1 = x[:, : E * LANES] + x[:, E * LANES :]
    y1 = y[:, : E * LANES] + y[:, E * LANES :]
    i = lax.broadcasted_iota(jnp.int32, (E * LANES, E), 0)
    j = lax.broadcasted_iota(jnp.int32, (E * LANES, E), 1)
    sel = jnp.where(i // LANES == j, 1.0, 0.0)
    cs = jnp.dot(x1, sel, preferred_element_type=jnp.float32)
    gs = jnp.dot(y1, sel, preferred_element_type=jnp.float32)
    out_ref[0, 0] = jnp.sum(cs * gs) * scale_ref[0]


def kernel(router_weights, router_logits, num_experts_per_tok, non_pad_token):
    del router_logits
    cnt_p, gsum_p = _sc_count_gsum(router_weights)
    valid = jnp.maximum(non_pad_token, 1)
    scale = (E / (valid * num_experts_per_tok)) / valid
    scale = jnp.asarray(scale, jnp.float32).reshape((1,))
    cnt2 = cnt_p.reshape(L_LAYERS, NC * E * LANES)
    gsum2 = gsum_p.reshape(L_LAYERS, NC * E * LANES)
    out = pl.pallas_call(
        _combine_body,
        out_shape=jax.ShapeDtypeStruct((1, 1), jnp.float32),
        in_specs=[
            pl.BlockSpec(memory_space=pltpu.SMEM),
            pl.BlockSpec(memory_space=pltpu.VMEM),
            pl.BlockSpec(memory_space=pltpu.VMEM),
        ],
        out_specs=pl.BlockSpec(memory_space=pltpu.SMEM),
    )(scale, cnt2, gsum2)
    return out[0, 0]
